# Initial kernel scaffold; baseline (speedup 1.0000x reference)
import jax
import jax.numpy as jnp
from jax.experimental import pallas as pl


def kernel(input_ids, table, W, b):
    raise NotImplementedError("write your pallas kernel here")


# speedup vs baseline: 8.4957x; 8.4957x over previous
import functools

import jax
import jax.numpy as jnp
from jax import lax
from jax.experimental import pallas as pl
from jax.experimental.pallas import tpu as pltpu
from jax.experimental.pallas import tpu_sc as plsc

_CTX = 50
_DIM = 32
_BATCH = 4096

_NC = 2
_NS = 16
_NW = _NC * _NS

_ROWS_PER_W = _BATCH // _NW
_G = 16
_NGRP = _ROWS_PER_W // _G
_STREAM = 100
_NSTREAM = _G * _CTX // _STREAM
_FEAT = _CTX * _DIM


def _sc_body(ids_ref, table_ref, ws_ref, b_ref, out_ref,
             idx_v, rows_v, ws_v, b_v, out_v, sem):
    wid = lax.axis_index("s") * _NC + lax.axis_index("c")

    pltpu.sync_copy(ids_ref.at[wid], idx_v)
    pltpu.sync_copy(ws_ref, ws_v)
    pltpu.sync_copy(b_ref, b_v)
    b_vec = b_v[pl.ds(0, 16)]
    lane_row = lax.iota(jnp.int32, 16) * _CTX

    cols = [jnp.full((16,), m, jnp.int32) for m in range(_DIM)]

    def group_body(g, _):
        copies = [
            pltpu.async_copy(
                table_ref.at[idx_v.at[g * _NSTREAM + j]],
                rows_v.at[pl.ds(j * _STREAM, _STREAM)],
                sem,
            )
            for j in range(_NSTREAM)
        ]
        for cp in copies:
            cp.wait()

        def c_body(c, acc):
            ridx = lane_row + c
            for m in range(_DIM):
                g16 = plsc.load_gather(rows_v, [ridx, cols[m]])
                w16 = ws_v[c * _DIM + m, pl.ds(0, 16)]
                acc = acc + g16 * w16
            return acc

        acc = lax.fori_loop(0, _CTX, c_body, b_vec)
        out_v[pl.ds(g * _G, _G)] = acc
        return 0

    lax.fori_loop(0, _NGRP, group_body, 0)

    pltpu.sync_copy(out_v, out_ref.at[pl.ds(wid * _ROWS_PER_W, _ROWS_PER_W)])


@jax.jit
def _logits_sc(ids, table, w_splat, b16):
    mesh = plsc.VectorSubcoreMesh(core_axis_name="c", subcore_axis_name="s")
    f = functools.partial(
        pl.kernel,
        out_type=jax.ShapeDtypeStruct((_BATCH,), jnp.float32),
        mesh=mesh,
        compiler_params=pltpu.CompilerParams(
            needs_layout_passes=False, use_tc_tiling_on_sc=False),
        scratch_types=[
            pltpu.VMEM((_ROWS_PER_W * _CTX // _STREAM, _STREAM), jnp.int32),
            pltpu.VMEM((_G * _CTX, _DIM), jnp.float32),
            pltpu.VMEM((_FEAT, 16), jnp.float32),
            pltpu.VMEM((16,), jnp.float32),
            pltpu.VMEM((_ROWS_PER_W,), jnp.float32),
            pltpu.SemaphoreType.DMA,
        ],
    )(_sc_body)
    return f(ids, table, w_splat, b16)


def kernel(input_ids, table, W, b):
    ids = input_ids.astype(jnp.int32).reshape(
        _NW, _ROWS_PER_W * _CTX // _STREAM, _STREAM)
    w_splat = jnp.broadcast_to(
        W.astype(jnp.float32).reshape(_FEAT, 1), (_FEAT, 16))
    b16 = jnp.broadcast_to(b.astype(jnp.float32), (16,))
    return _logits_sc(ids, table.astype(jnp.float32), w_splat, b16)
